# baseline (speedup 1.0000x reference)
import functools

import jax
import jax.numpy as jnp
from jax import lax
from jax.experimental import pallas as pl
from jax.experimental.pallas import tpu as pltpu
from jax.experimental.pallas import tpu_sc as plsc

EMB = 32
BB = 128
NRING = 4
TILE = 8 * 128


@functools.cache
def _build(L, NW):
    mesh = plsc.VectorSubcoreMesh(core_axis_name="c", subcore_axis_name="s")
    info = plsc.get_sparse_core_info()
    NC = info.num_cores
    assert L % NRING == 0

    @functools.partial(
        pl.kernel,
        out_type=jax.ShapeDtypeStruct((L, 4, NW, TILE), jnp.float32),
        mesh=mesh,
        scratch_types=[
            pltpu.VMEM((L, BB), jnp.int32),
            *[pltpu.VMEM((BB, EMB), jnp.float32) for _ in range(NRING)],
            *[pltpu.VMEM((BB * EMB,), jnp.float32) for _ in range(2)],
            *[pltpu.SemaphoreType.DMA for _ in range(NRING + 2)],
        ],
        compiler_params=pltpu.CompilerParams(
            use_tc_tiling_on_sc=False, needs_layout_passes=False),
    )
    def k(idx_hbm, table_hbm, out_hbm,
          idx_v, rv0, rv1, rv2, rv3, rt0, rt1,
          sg0, sg1, sg2, sg3, sw0, sw1):
        ring = (rv0, rv1, rv2, rv3)
        sg = (sg0, sg1, sg2, sg3)
        rts = (rt0, rt1)
        sws = (sw0, sw1)
        wid = lax.axis_index("s") * NC + lax.axis_index("c")
        pltpu.sync_copy(idx_hbm.at[wid], idx_v)

        ebase0 = lax.broadcasted_iota(jnp.int32, (16,), 0) * BB
        ebase1 = ebase0 + 16 * BB

        def fire_gather(l, slot):
            pltpu.async_copy(table_hbm.at[idx_v.at[l]], ring[slot], sg[slot])

        def wait_gather(slot):
            pltpu.make_async_copy(
                table_hbm.at[pl.ds(0, BB)], ring[slot], sg[slot]).wait()

        def transpose_block(rv, rt):
            def tbody(b0, carry):
                for u in range(8):
                    b = b0 * 8 + u
                    v0 = rv[b, pl.ds(0, 16)]
                    v1 = rv[b, pl.ds(16, 16)]
                    plsc.store_scatter(rt, [ebase0 + b], v0)
                    plsc.store_scatter(rt, [ebase1 + b], v1)
                return carry
            lax.fori_loop(0, BB // 8, tbody, 0)

        def fire_writeout(l, rt, sem):
            for eb in range(4):
                pltpu.async_copy(
                    rt.at[pl.ds(eb * TILE, TILE)],
                    out_hbm.at[l, eb, wid],
                    sem)

        def wait_writeout(rt, sem):
            for eb in range(4):
                pltpu.make_async_copy(
                    rt.at[pl.ds(eb * TILE, TILE)],
                    out_hbm.at[0, eb, wid],
                    sem).wait()

        for r in range(NRING - 1):
            fire_gather(r, r)

        for r in range(NRING):
            l = r
            s2 = r % 2
            fire_gather(l + NRING - 1, (r + NRING - 1) % NRING)
            wait_gather(r)
            if l >= 2:
                wait_writeout(rts[s2], sws[s2])
            transpose_block(ring[r], rts[s2])
            fire_writeout(l, rts[s2], sws[s2])

        def trip(t, carry):
            for r in range(NRING):
                l = t * NRING + r
                s2 = r % 2
                fire_gather(l + NRING - 1, (r + NRING - 1) % NRING)
                wait_gather(r)
                wait_writeout(rts[s2], sws[s2])
                transpose_block(ring[r], rts[s2])
                fire_writeout(l, rts[s2], sws[s2])
            return carry

        lax.fori_loop(1, L // NRING - 1, trip, 0)

        for r in range(NRING):
            l = L - NRING + r
            s2 = r % 2
            if r == 0:
                fire_gather(L - 1, (L - 1) % NRING)
            wait_gather(r)
            wait_writeout(rts[s2], sws[s2])
            transpose_block(ring[r], rts[s2])
            fire_writeout(l, rts[s2], sws[s2])
        for s2 in range(2):
            wait_writeout(rts[s2], sws[s2])

    return k


def kernel(inputs, embedding_table):
    B_, L_ = inputs.shape
    NW = B_ // BB
    idx3 = inputs.T.reshape(L_, NW, BB).transpose(1, 0, 2).astype(jnp.int32)
    out5 = _build(L_, NW)(idx3, embedding_table)
    out = (out5.reshape(L_, 4, NW, 8, BB)
           .transpose(2, 4, 0, 1, 3)
           .reshape(B_, L_, EMB))
    return out, embedding_table
